# baseline (speedup 1.0000x reference)
import jax
import jax.numpy as jnp
from jax.experimental import pallas as pl
from jax.experimental.pallas import tpu as pltpu

_NE = 1024
_D = 64
_R = 512
_N = 16 * 1024
_CCOST = 0.25


def _vq_body(x_ref, w_ref, q_ref, idx_ref, md_ref, loss_ref, ppl_ref,
             w2_ref, usage_ref, acc_ref):
    i = pl.program_id(0)

    @pl.when(i == 0)
    def _init():
        w = w_ref[...]
        w2_ref[...] = jnp.sum(w * w, axis=1)[None, :]
        usage_ref[...] = jnp.zeros_like(usage_ref)
        acc_ref[0] = 0.0
        acc_ref[1] = 0.0

    x = x_ref[...]
    w = w_ref[...]
    x2 = jnp.sum(x * x, axis=1, keepdims=True)
    dots = jax.lax.dot_general(-2.0 * x, w, (((1,), (1,)), ((), ())),
                               preferred_element_type=jnp.float32)
    dist = dots + w2_ref[...]
    mind0 = jnp.min(dist, axis=1, keepdims=True)
    cols = jax.lax.broadcasted_iota(jnp.int32, dist.shape, 1)
    amin = jnp.min(jnp.where(dist == mind0, cols, _NE), axis=1,
                   keepdims=True)
    mind = mind0 + x2
    validk = jnp.sqrt(x2) > 1e-6
    maskf = validk.astype(jnp.float32)
    oh = (cols == amin).astype(jnp.float32)
    qa = jax.lax.dot_general(oh, w, (((1,), (0,)), ((), ())),
                             preferred_element_type=jnp.float32)
    q_ref[...] = qa * maskf
    idx_ref[...] = jnp.where(validk, amin, 0)[:, 0][None, None, :]
    md_ref[...] = jnp.where(validk, mind, 0.0)[:, 0][None, None, :]
    diff = x - qa
    ohm = oh * maskf
    usage_ref[...] += jnp.sum(ohm.reshape(_R // 8, 8, _NE), axis=0)
    acc_ref[0] += jnp.sum(diff * diff * maskf)
    acc_ref[1] += jnp.sum(maskf)

    @pl.when(i == pl.num_programs(0) - 1)
    def _fini():
        nv = jnp.maximum(acc_ref[1], 1.0)
        loss_ref[...] = jnp.full((1, 1), _CCOST / _D) * (acc_ref[0] / nv)
        avg = jnp.sum(usage_ref[...], axis=0)[None, :] / nv
        ent = -jnp.sum(avg * jnp.log(avg + 1e-10))
        ppl_ref[...] = jnp.exp(jnp.full((1, 1), 1.0) * ent)


_GRID = _N // _R

_vq_call = pl.pallas_call(
    _vq_body,
    grid=(_GRID,),
    in_specs=[pl.BlockSpec((_R, _D), lambda i: (i, 0)),
              pl.BlockSpec((_NE, _D), lambda i: (0, 0))],
    out_specs=[pl.BlockSpec((_R, _D), lambda i: (i, 0)),
               pl.BlockSpec((1, 1, _R), lambda i: (i, 0, 0)),
               pl.BlockSpec((1, 1, _R), lambda i: (i, 0, 0)),
               pl.BlockSpec((1, 1), lambda i: (0, 0)),
               pl.BlockSpec((1, 1), lambda i: (0, 0))],
    out_shape=[
        jax.ShapeDtypeStruct((_N, _D), jnp.float32),
        jax.ShapeDtypeStruct((_GRID, 1, _R), jnp.int32),
        jax.ShapeDtypeStruct((_GRID, 1, _R), jnp.float32),
        jax.ShapeDtypeStruct((1, 1), jnp.float32),
        jax.ShapeDtypeStruct((1, 1), jnp.float32),
    ],
    scratch_shapes=[pltpu.VMEM((1, _NE), jnp.float32),
                    pltpu.VMEM((8, _NE), jnp.float32),
                    pltpu.SMEM((2,), jnp.float32)],
)


def kernel(inputs, W):
    shape = inputs.shape
    flat = inputs.reshape(-1, _D)
    q, idx, md, loss, ppl = _vq_call(flat, W)
    quantized = q.reshape(shape)
    indices = idx.reshape(shape[:-1])
    min_distances = md.reshape(shape[:-1])
    return (quantized, loss[0, 0], ppl[0, 0], indices, min_distances)


# speedup vs baseline: 1.0400x; 1.0400x over previous
import jax
import jax.numpy as jnp
from jax.experimental import pallas as pl
from jax.experimental.pallas import tpu as pltpu

_NE = 1024
_D = 64
_R = 512
_N = 16 * 1024
_CCOST = 0.25


def _vq_body(x_ref, w_ref, q_ref, idx_ref, md_ref, loss_ref, ppl_ref,
             w2_ref, usage_ref, acc_ref):
    i = pl.program_id(0)

    @pl.when(i == 0)
    def _init():
        w = w_ref[...]
        w2_ref[...] = jnp.sum(w * w, axis=1)[None, :]
        usage_ref[...] = jnp.zeros_like(usage_ref)
        acc_ref[0] = 0.0
        acc_ref[1] = 0.0

    x = x_ref[...]
    w = w_ref[...]
    x2 = jnp.sum(x * x, axis=1, keepdims=True)
    dots = jax.lax.dot_general(-2.0 * x, w, (((1,), (1,)), ((), ())),
                               preferred_element_type=jnp.float32)
    dist = dots + w2_ref[...]
    mind0 = jnp.min(dist, axis=1, keepdims=True)
    cols = jax.lax.broadcasted_iota(jnp.int32, dist.shape, 1)
    amin = jnp.min(jnp.where(dist == mind0, cols, _NE), axis=1,
                   keepdims=True)
    mind = mind0 + x2
    validk = jnp.sqrt(x2) > 1e-6
    maskf = validk.astype(jnp.float32)
    amin_g = jnp.where(validk, amin, _NE)
    oh = (cols == amin_g).astype(jnp.float32)
    qa = jax.lax.dot_general(oh, w, (((1,), (0,)), ((), ())),
                             preferred_element_type=jnp.float32)
    q_ref[...] = qa
    idx_ref[...] = jnp.where(validk, amin, 0)[:, 0][None, None, :]
    md_ref[...] = jnp.where(validk, mind, 0.0)[:, 0][None, None, :]
    diff = x - qa
    usage_ref[...] += jnp.sum(oh.reshape(_R // 8, 8, _NE), axis=0)
    acc_ref[0] += jnp.sum(diff * diff * maskf)
    acc_ref[1] += jnp.sum(maskf)

    @pl.when(i == pl.num_programs(0) - 1)
    def _fini():
        nv = jnp.maximum(acc_ref[1], 1.0)
        loss_ref[...] = jnp.full((1, 1), _CCOST / _D) * (acc_ref[0] / nv)
        avg = jnp.sum(usage_ref[...], axis=0)[None, :] / nv
        ent = -jnp.sum(avg * jnp.log(avg + 1e-10))
        ppl_ref[...] = jnp.exp(jnp.full((1, 1), 1.0) * ent)


_GRID = _N // _R

_vq_call = pl.pallas_call(
    _vq_body,
    grid=(_GRID,),
    in_specs=[pl.BlockSpec((_R, _D), lambda i: (i, 0)),
              pl.BlockSpec((_NE, _D), lambda i: (0, 0))],
    out_specs=[pl.BlockSpec((_R, _D), lambda i: (i, 0)),
               pl.BlockSpec((1, 1, _R), lambda i: (i, 0, 0)),
               pl.BlockSpec((1, 1, _R), lambda i: (i, 0, 0)),
               pl.BlockSpec((1, 1), lambda i: (0, 0)),
               pl.BlockSpec((1, 1), lambda i: (0, 0))],
    out_shape=[
        jax.ShapeDtypeStruct((_N, _D), jnp.float32),
        jax.ShapeDtypeStruct((_GRID, 1, _R), jnp.int32),
        jax.ShapeDtypeStruct((_GRID, 1, _R), jnp.float32),
        jax.ShapeDtypeStruct((1, 1), jnp.float32),
        jax.ShapeDtypeStruct((1, 1), jnp.float32),
    ],
    scratch_shapes=[pltpu.VMEM((1, _NE), jnp.float32),
                    pltpu.VMEM((8, _NE), jnp.float32),
                    pltpu.SMEM((2,), jnp.float32)],
)


def kernel(inputs, W):
    shape = inputs.shape
    flat = inputs.reshape(-1, _D)
    q, idx, md, loss, ppl = _vq_call(flat, W)
    quantized = q.reshape(shape)
    indices = idx.reshape(shape[:-1])
    min_distances = md.reshape(shape[:-1])
    return (quantized, loss[0, 0], ppl[0, 0], indices, min_distances)


# speedup vs baseline: 1.0777x; 1.0362x over previous
import jax
import jax.numpy as jnp
from jax.experimental import pallas as pl
from jax.experimental.pallas import tpu as pltpu

_NE = 1024
_D = 64
_R = 512
_N = 16 * 1024
_CCOST = 0.25


def _vq_body(x_ref, w_ref, q_ref, idx_ref, md_ref, loss_ref, ppl_ref,
             w2_ref, usage_ref, acc_ref):
    i = pl.program_id(0)

    @pl.when(i == 0)
    def _init():
        w = w_ref[...]
        w2_ref[...] = jnp.sum(w * w, axis=1)[None, :]
        usage_ref[...] = jnp.zeros_like(usage_ref)
        acc_ref[0] = 0.0
        acc_ref[1] = 0.0

    x = x_ref[...]
    w = w_ref[...]
    x2 = jnp.sum(x * x, axis=1, keepdims=True)
    dots = jax.lax.dot_general(-2.0 * x, w, (((1,), (1,)), ((), ())),
                               preferred_element_type=jnp.float32)
    dist = dots + w2_ref[...]
    mind0 = jnp.min(dist, axis=1, keepdims=True)
    colsf = jax.lax.broadcasted_iota(jnp.int32, (1, _NE), 1).astype(jnp.float32)
    aminf = jnp.min(jnp.where(dist == mind0, colsf, float(_NE)), axis=1,
                    keepdims=True)
    mind = mind0 + x2
    validk = jnp.sqrt(x2) > 1e-6
    maskf = validk.astype(jnp.float32)
    aminf_g = jnp.where(validk, aminf, float(_NE))
    oh = (colsf == aminf_g).astype(jnp.float32)
    qa = jax.lax.dot_general(oh, w, (((1,), (0,)), ((), ())),
                             preferred_element_type=jnp.float32)
    q_ref[...] = qa
    amin = aminf.astype(jnp.int32)
    idx_ref[...] = jnp.where(validk, amin, 0)[:, 0][None, None, :]
    md_ref[...] = jnp.where(validk, mind, 0.0)[:, 0][None, None, :]
    diff = x - qa
    usage_ref[...] += jnp.sum(oh.reshape(_R // 8, 8, _NE), axis=0)
    acc_ref[0] += jnp.sum(diff * diff * maskf)
    acc_ref[1] += jnp.sum(maskf)

    @pl.when(i == pl.num_programs(0) - 1)
    def _fini():
        nv = jnp.maximum(acc_ref[1], 1.0)
        loss_ref[...] = jnp.full((1, 1), _CCOST / _D) * (acc_ref[0] / nv)
        avg = jnp.sum(usage_ref[...], axis=0)[None, :] / nv
        ent = -jnp.sum(avg * jnp.log(avg + 1e-10))
        ppl_ref[...] = jnp.exp(jnp.full((1, 1), 1.0) * ent)


_GRID = _N // _R

_vq_call = pl.pallas_call(
    _vq_body,
    grid=(_GRID,),
    in_specs=[pl.BlockSpec((_R, _D), lambda i: (i, 0)),
              pl.BlockSpec((_NE, _D), lambda i: (0, 0))],
    out_specs=[pl.BlockSpec((_R, _D), lambda i: (i, 0)),
               pl.BlockSpec((1, 1, _R), lambda i: (i, 0, 0)),
               pl.BlockSpec((1, 1, _R), lambda i: (i, 0, 0)),
               pl.BlockSpec((1, 1), lambda i: (0, 0)),
               pl.BlockSpec((1, 1), lambda i: (0, 0))],
    out_shape=[
        jax.ShapeDtypeStruct((_N, _D), jnp.float32),
        jax.ShapeDtypeStruct((_GRID, 1, _R), jnp.int32),
        jax.ShapeDtypeStruct((_GRID, 1, _R), jnp.float32),
        jax.ShapeDtypeStruct((1, 1), jnp.float32),
        jax.ShapeDtypeStruct((1, 1), jnp.float32),
    ],
    scratch_shapes=[pltpu.VMEM((1, _NE), jnp.float32),
                    pltpu.VMEM((8, _NE), jnp.float32),
                    pltpu.SMEM((2,), jnp.float32)],
)


def kernel(inputs, W):
    shape = inputs.shape
    flat = inputs.reshape(-1, _D)
    q, idx, md, loss, ppl = _vq_call(flat, W)
    quantized = q.reshape(shape)
    indices = idx.reshape(shape[:-1])
    min_distances = md.reshape(shape[:-1])
    return (quantized, loss[0, 0], ppl[0, 0], indices, min_distances)
